# Initial kernel scaffold; baseline (speedup 1.0000x reference)
import jax
import jax.numpy as jnp
from jax.experimental import pallas as pl


def kernel(w1, b1, w2, b2, wf1t, bf1, wf2t, bf2, x):
    raise NotImplementedError("write your pallas kernel here")


# speedup vs baseline: 15.3610x; 15.3610x over previous
import jax
import jax.numpy as jnp
from jax.experimental import pallas as pl
from jax.experimental.pallas import tpu as pltpu

_VMEM_LIMIT = 64 * 1024 * 1024


def _build_a1(w1):
    iw = jnp.arange(28)
    cols = jnp.arange(240)
    rw = cols // 120
    oc = (cols % 120) // 12
    pw = cols % 12
    ow = 2 * pw + rw
    d = iw[:, None] - ow[None, :]
    mask = (d >= 0) & (d < 5)
    kh = jnp.arange(5)
    idx = kh[:, None, None] * 5 + jnp.clip(d, 0, 4)[None]
    vals = w1[oc[None, None, :], idx]
    return jnp.where(mask[None], vals, 0.0)


def _build_a2(w2):
    rows = jnp.arange(120)
    c = rows // 12
    iw = rows % 12
    cols = jnp.arange(200)
    oc = cols // 10
    ow = cols % 10
    d = iw[:, None] - ow[None, :]
    mask = (d >= 0) & (d < 3)
    kh = jnp.arange(3)
    idx = c[None, :, None] * 9 + kh[:, None, None] * 3 + jnp.clip(d, 0, 2)[None]
    vals = w2[oc[None, None, :], idx]
    return jnp.where(mask[None], vals, 0.0)


def _lenet_kernel(x_ref, a1_ref, b1c_ref, a2_ref, b2c_ref, w1p_ref, bf1_ref,
                  w2t_ref, bf2_ref, o_ref):
    tb = x_ref.shape[0]
    x3 = x_ref[...].reshape(tb, 28, 28)
    acc = None
    for kh in range(5):
        xk = x3[:, kh:kh + 24, :].reshape(tb * 24, 28)
        d = jnp.dot(xk, a1_ref[kh], preferred_element_type=jnp.float32)
        acc = d if acc is None else acc + d
    acc = jnp.maximum(acc + b1c_ref[...], 0.0)
    hp = jnp.maximum(acc[:, :120], acc[:, 120:])
    vp = jnp.max(hp.reshape(tb * 12, 2, 120), axis=1)
    pt = vp.reshape(tb, 12, 120)
    acc2 = None
    for kh in range(3):
        pk = pt[:, kh:kh + 10, :].reshape(tb * 10, 120)
        d = jnp.dot(pk, a2_ref[kh], preferred_element_type=jnp.float32)
        acc2 = d if acc2 is None else acc2 + d
    acc2 = jnp.maximum(acc2 + b2c_ref[...], 0.0)
    acc2r = acc2.reshape(tb, 10, 200)
    h = None
    for oh in range(10):
        d = jnp.dot(acc2r[:, oh, :], w1p_ref[oh],
                    preferred_element_type=jnp.float32)
        h = d if h is None else h + d
    h = jnp.maximum(h + bf1_ref[...], 0.0)
    logits = jnp.dot(h, w2t_ref[...], preferred_element_type=jnp.float32)
    logits = logits + bf2_ref[...]
    m = jnp.max(logits, axis=-1, keepdims=True)
    s = logits - m
    lse = jnp.log(jnp.sum(jnp.exp(s), axis=-1, keepdims=True))
    o_ref[...] = (s - lse).astype(o_ref.dtype)


def kernel(w1, b1, w2, b2, wf1t, bf1, wf2t, bf2, x):
    batch = x.shape[0]
    tb = 128 if batch % 128 == 0 else batch
    xf = x.reshape(batch, 28 * 28)
    a1 = _build_a1(w1)
    a2 = _build_a2(w2)
    b1c = jnp.tile(jnp.repeat(b1.reshape(10), 12), 2).reshape(1, 240)
    b2c = jnp.repeat(b2.reshape(20), 10).reshape(1, 200)
    w1p = wf1t.reshape(20, 10, 10, 500).transpose(1, 0, 2, 3).reshape(10, 200, 500)
    cost = pl.CostEstimate(
        flops=2 * batch * (5 * 24 * 28 * 240 + 3 * 10 * 120 * 200
                           + 2000 * 500 + 500 * 10),
        transcendentals=batch * 11,
        bytes_accessed=4 * (xf.size + batch * 10 + a1.size + a2.size
                            + w1p.size + wf2t.size),
    )
    return pl.pallas_call(
        _lenet_kernel,
        out_shape=jax.ShapeDtypeStruct((batch, 10), x.dtype),
        grid=(batch // tb,),
        in_specs=[
            pl.BlockSpec((tb, 784), lambda i: (i, 0)),
            pl.BlockSpec((5, 28, 240), lambda i: (0, 0, 0)),
            pl.BlockSpec((1, 240), lambda i: (0, 0)),
            pl.BlockSpec((3, 120, 200), lambda i: (0, 0, 0)),
            pl.BlockSpec((1, 200), lambda i: (0, 0)),
            pl.BlockSpec((10, 200, 500), lambda i: (0, 0, 0)),
            pl.BlockSpec((1, 500), lambda i: (0, 0)),
            pl.BlockSpec((500, 10), lambda i: (0, 0)),
            pl.BlockSpec((1, 10), lambda i: (0, 0)),
        ],
        out_specs=pl.BlockSpec((tb, 10), lambda i: (i, 0)),
        compiler_params=pltpu.CompilerParams(
            dimension_semantics=("parallel",),
            vmem_limit_bytes=_VMEM_LIMIT,
        ),
        cost_estimate=cost,
    )(xf, a1, b1c, a2, b2c, w1p, bf1, wf2t, bf2)


# speedup vs baseline: 37.9308x; 2.4693x over previous
import jax
import jax.numpy as jnp
import numpy as np
from jax.experimental import pallas as pl
from jax.experimental.pallas import tpu as pltpu

_VMEM_LIMIT = 64 * 1024 * 1024


def _oh1():
    iw = np.arange(28)[None, :, None]
    col = np.arange(240)[None, None, :]
    ow = 2 * (col % 12) + col // 120
    kw = np.arange(5)[:, None, None]
    return (iw - ow == kw).astype(np.float32)


def _oh2():
    iw = np.arange(120)[None, :, None] % 12
    col = np.arange(200)[None, None, :]
    ow = col % 10
    kw = np.arange(3)[:, None, None]
    return (iw - ow == kw).astype(np.float32)


_OH1 = _oh1()
_OH2 = _oh2()


def _build_a1(w1):
    w1k = w1.reshape(10, 5, 5).transpose(1, 2, 0)
    e1 = jnp.tile(jnp.repeat(w1k, 12, axis=2), (1, 1, 2))
    return jnp.einsum("hwc,wic->hic", e1, _OH1)


def _build_a2(w2):
    w2k = w2.reshape(20, 10, 3, 3).transpose(2, 3, 1, 0)
    e2 = jnp.repeat(jnp.repeat(w2k, 12, axis=2), 10, axis=3)
    return jnp.einsum("hwrc,wrc->hrc", e2, _OH2)


def _lenet_kernel(x_ref, a1_ref, b1c_ref, a2_ref, b2c_ref, w1p_ref, bf1_ref,
                  w2t_ref, bf2_ref, o_ref):
    tb = x_ref.shape[0]
    x3 = x_ref[...].reshape(tb, 28, 28)
    acc = None
    for kh in range(5):
        xk = x3[:, kh:kh + 24, :].reshape(tb * 24, 28)
        d = jnp.dot(xk, a1_ref[kh], preferred_element_type=jnp.float32)
        acc = d if acc is None else acc + d
    acc = jnp.maximum(acc + b1c_ref[...], 0.0)
    hp = jnp.maximum(acc[:, :120], acc[:, 120:])
    vp = jnp.max(hp.reshape(tb * 12, 2, 120), axis=1)
    pt = vp.reshape(tb, 12, 120)
    acc2 = None
    for kh in range(3):
        pk = pt[:, kh:kh + 10, :].reshape(tb * 10, 120)
        d = jnp.dot(pk, a2_ref[kh], preferred_element_type=jnp.float32)
        acc2 = d if acc2 is None else acc2 + d
    acc2 = jnp.maximum(acc2 + b2c_ref[...], 0.0)
    acc2r = acc2.reshape(tb, 10, 200)
    h = None
    for oh in range(10):
        d = jnp.dot(acc2r[:, oh, :], w1p_ref[oh],
                    preferred_element_type=jnp.float32)
        h = d if h is None else h + d
    h = jnp.maximum(h + bf1_ref[...], 0.0)
    logits = jnp.dot(h, w2t_ref[...], preferred_element_type=jnp.float32)
    logits = logits + bf2_ref[...]
    m = jnp.max(logits, axis=-1, keepdims=True)
    s = logits - m
    lse = jnp.log(jnp.sum(jnp.exp(s), axis=-1, keepdims=True))
    o_ref[...] = (s - lse).astype(o_ref.dtype)


def kernel(w1, b1, w2, b2, wf1t, bf1, wf2t, bf2, x):
    batch = x.shape[0]
    tb = 128 if batch % 128 == 0 else batch
    xf = x.reshape(batch, 28 * 28)
    a1 = _build_a1(w1)
    a2 = _build_a2(w2)
    b1c = jnp.tile(jnp.repeat(b1.reshape(10), 12), 2).reshape(1, 240)
    b2c = jnp.repeat(b2.reshape(20), 10).reshape(1, 200)
    w1p = wf1t.reshape(20, 10, 10, 500).transpose(1, 0, 2, 3).reshape(10, 200, 500)
    cost = pl.CostEstimate(
        flops=2 * batch * (5 * 24 * 28 * 240 + 3 * 10 * 120 * 200
                           + 2000 * 500 + 500 * 10),
        transcendentals=batch * 11,
        bytes_accessed=4 * (xf.size + batch * 10 + a1.size + a2.size
                            + w1p.size + wf2t.size),
    )
    return pl.pallas_call(
        _lenet_kernel,
        out_shape=jax.ShapeDtypeStruct((batch, 10), x.dtype),
        grid=(batch // tb,),
        in_specs=[
            pl.BlockSpec((tb, 784), lambda i: (i, 0)),
            pl.BlockSpec((5, 28, 240), lambda i: (0, 0, 0)),
            pl.BlockSpec((1, 240), lambda i: (0, 0)),
            pl.BlockSpec((3, 120, 200), lambda i: (0, 0, 0)),
            pl.BlockSpec((1, 200), lambda i: (0, 0)),
            pl.BlockSpec((10, 200, 500), lambda i: (0, 0, 0)),
            pl.BlockSpec((1, 500), lambda i: (0, 0)),
            pl.BlockSpec((500, 10), lambda i: (0, 0)),
            pl.BlockSpec((1, 10), lambda i: (0, 0)),
        ],
        out_specs=pl.BlockSpec((tb, 10), lambda i: (i, 0)),
        compiler_params=pltpu.CompilerParams(
            dimension_semantics=("parallel",),
            vmem_limit_bytes=_VMEM_LIMIT,
        ),
        cost_estimate=cost,
    )(xf, a1, b1c, a2, b2c, w1p, bf1, wf2t, bf2)
